# baseline (speedup 1.0000x reference)
import functools

import jax
import jax.numpy as jnp
from jax import lax
from jax.experimental import pallas as pl
from jax.experimental.pallas import tpu as pltpu
from jax.experimental.pallas import tpu_sc as plsc

NC = 2
NS = 16
NW = NC * NS
L = 16

CHUNK = 1024
GFAN = 128
TW = 8


def _make_kernel(n, h, w, c):
    per_worker = n // NW
    n_chunks = per_worker // CHUNK
    assert per_worker % CHUNK == 0 and CHUNK % (8 * L) == 0
    assert (w * c) % TW == 0
    krow = w * c // TW

    mesh = plsc.VectorSubcoreMesh(
        core_axis_name="c", subcore_axis_name="s",
        num_cores=NC, num_subcores=NS)

    @functools.partial(
        pl.kernel,
        out_type=jax.ShapeDtypeStruct((n * c,), jnp.float32),
        mesh=mesh,
        scratch_types=dict(
            xs_v=pltpu.VMEM((CHUNK * 2,), jnp.float32),
            ib0=pltpu.VMEM((CHUNK * 2,), jnp.int32),
            ib1=pltpu.VMEM((CHUNK * 2,), jnp.int32),
            ov=pltpu.VMEM((CHUNK,), jnp.int32),
            fxv=pltpu.VMEM((CHUNK,), jnp.float32),
            fyv=pltpu.VMEM((CHUNK,), jnp.float32),
            t0=pltpu.VMEM((CHUNK * 2, TW), jnp.float32),
            t1=pltpu.VMEM((CHUNK * 2, TW), jnp.float32),
            out_v=pltpu.VMEM((CHUNK * c,), jnp.float32),
            sem=pltpu.SemaphoreType.DMA,
        ),
        compiler_params=pltpu.CompilerParams(
            needs_layout_passes=False, use_tc_tiling_on_sc=False),
    )
    def image_kernel(xs_hbm, table_hbm, out_hbm, *, xs_v,
                     ib0, ib1, ov, fxv, fyv, t0, t1, out_v, sem):
        wid = lax.axis_index("s") * NC + lax.axis_index("c")
        base0 = wid * per_worker
        wf = jnp.float32(w)
        hf = jnp.float32(h)

        def chunk_body(ci, carry):
            base = base0 + ci * CHUNK
            pltpu.sync_copy(xs_hbm.at[pl.ds(base * 2, CHUNK * 2)], xs_v)

            iota = lax.iota(jnp.int32, L)

            def phase1(i, carry):
                p = i * L + iota
                p2 = p * 2
                x = plsc.load_gather(xs_v, [p2])
                y = plsc.load_gather(xs_v, [p2 + 1])
                sx = x * wf
                sy = y * hf
                xi = sx.astype(jnp.int32)
                yi = sy.astype(jnp.int32)
                fx = sx - xi.astype(jnp.float32)
                fy = sy - yi.astype(jnp.float32)
                x0 = jnp.clip(xi, 0, w - 1)
                y0 = jnp.clip(yi, 0, h - 1)
                fx = jnp.where(x0 == w - 1, jnp.float32(1.0), fx)
                fy = jnp.where(y0 == h - 1, jnp.float32(1.0), fy)
                x0 = jnp.minimum(x0, w - 2)
                y0 = jnp.minimum(y0, h - 2)
                b0 = (y0 * w + x0) * c
                k0 = jnp.right_shift(b0, 3)
                sl = pl.ds(i * L, L)
                plsc.store_scatter(ib0, [p2], k0)
                plsc.store_scatter(ib0, [p2 + 1], k0 + 1)
                plsc.store_scatter(ib1, [p2], k0 + krow)
                plsc.store_scatter(ib1, [p2 + 1], k0 + krow + 1)
                ov[sl] = b0 & 7
                fxv[sl] = fx
                fyv[sl] = fy
                return carry

            lax.fori_loop(0, CHUNK // L, phase1, 0, unroll=2)

            descs = []
            for j in range(CHUNK * 2 // GFAN):
                sl = pl.ds(j * GFAN, GFAN)
                for iv, tv in ((ib0, t0), (ib1, t1)):
                    descs.append(
                        pltpu.async_copy(table_hbm.at[iv.at[sl]], tv.at[sl], sem))
            for d in descs:
                d.wait()

            def phase3(i, carry):
                p = i * L + iota
                pc = p * c
                p2 = p * 2
                sl = pl.ds(i * L, L)
                o = ov[sl]
                fx = fxv[sl]
                fy = fyv[sl]
                gx = 1.0 - fx
                gy = 1.0 - fy
                for ch in range(c):
                    qa = o + ch
                    qb = qa + c
                    ra = p2 + jnp.right_shift(qa, 3)
                    ca = qa & 7
                    rb = p2 + jnp.right_shift(qb, 3)
                    cb = qb & 7
                    top = (plsc.load_gather(t0, [ra, ca]) * gx
                           + plsc.load_gather(t0, [rb, cb]) * fx)
                    bot = (plsc.load_gather(t1, [ra, ca]) * gx
                           + plsc.load_gather(t1, [rb, cb]) * fx)
                    plsc.store_scatter(out_v, [pc + ch], top * gy + bot * fy)
                return carry

            lax.fori_loop(0, CHUNK // L, phase3, 0, unroll=2)

            pltpu.sync_copy(out_v, out_hbm.at[pl.ds(base * c, CHUNK * c)])
            return carry

        lax.fori_loop(0, n_chunks, chunk_body, 0)

    return image_kernel


@jax.jit
def kernel(xs, data):
    h, w, c = data.shape
    n = xs.shape[0]
    flat = data.reshape(-1)
    half = flat.shape[0] // 2
    table = jnp.concatenate(
        [flat[:half].reshape(-1, TW), flat[half:].reshape(-1, TW)], axis=0)
    out_flat = _make_kernel(n, h, w, c)(xs.reshape(-1), table)
    return out_flat.reshape(n, c)


# speedup vs baseline: 1.5715x; 1.5715x over previous
import functools

import jax
import jax.numpy as jnp
from jax import lax
from jax.experimental import pallas as pl
from jax.experimental.pallas import tpu as pltpu
from jax.experimental.pallas import tpu_sc as plsc

NC = 2
NS = 16
NW = NC * NS
L = 16

CHUNK = 1024
GFAN = 128
TW = 8
BLK = 8192

_CP = pltpu.CompilerParams(
    needs_layout_passes=False, use_tc_tiling_on_sc=False)
_MESH = plsc.VectorSubcoreMesh(
    core_axis_name="c", subcore_axis_name="s",
    num_cores=NC, num_subcores=NS)


def _make_relayout(total):
    rows = total // TW
    per_worker = total // NW
    n_blocks = per_worker // BLK
    assert per_worker % BLK == 0

    @functools.partial(
        pl.kernel,
        out_type=jax.ShapeDtypeStruct((rows, TW), jnp.float32),
        mesh=_MESH,
        scratch_types=dict(
            vbuf=pltpu.VMEM((BLK,), jnp.float32),
            wbuf=pltpu.VMEM((BLK // TW, TW), jnp.float32),
        ),
        compiler_params=_CP,
    )
    def relayout_kernel(img_hbm, out_hbm, *, vbuf, wbuf):
        wid = lax.axis_index("s") * NC + lax.axis_index("c")
        base0 = wid * per_worker
        iota = lax.iota(jnp.int32, L)
        rv_pat = jnp.right_shift(iota, 3)
        cv = iota & 7

        def block_body(bi, carry):
            base = base0 + bi * BLK
            pltpu.sync_copy(img_hbm.at[pl.ds(base, BLK)], vbuf)

            def mv(j, carry):
                v = vbuf[pl.ds(j * L, L)]
                plsc.store_scatter(wbuf, [j * 2 + rv_pat, cv], v)
                return carry

            lax.fori_loop(0, BLK // L, mv, 0, unroll=4)
            pltpu.sync_copy(wbuf, out_hbm.at[pl.ds(base // TW, BLK // TW)])
            return carry

        lax.fori_loop(0, n_blocks, block_body, 0)

    return relayout_kernel


def _make_kernel(n, h, w, c):
    per_worker = n // NW
    n_chunks = per_worker // CHUNK
    assert per_worker % CHUNK == 0 and CHUNK % (8 * L) == 0
    assert (w * c) % TW == 0
    krow = w * c // TW

    @functools.partial(
        pl.kernel,
        out_type=jax.ShapeDtypeStruct((n * c,), jnp.float32),
        mesh=_MESH,
        scratch_types=dict(
            xs_v=pltpu.VMEM((CHUNK * 2,), jnp.float32),
            ib0=pltpu.VMEM((CHUNK * 2,), jnp.int32),
            ib1=pltpu.VMEM((CHUNK * 2,), jnp.int32),
            ov=pltpu.VMEM((CHUNK,), jnp.int32),
            fxv=pltpu.VMEM((CHUNK,), jnp.float32),
            fyv=pltpu.VMEM((CHUNK,), jnp.float32),
            t0=pltpu.VMEM((CHUNK * 2, TW), jnp.float32),
            t1=pltpu.VMEM((CHUNK * 2, TW), jnp.float32),
            out_v=pltpu.VMEM((CHUNK * c,), jnp.float32),
            sem=pltpu.SemaphoreType.DMA,
        ),
        compiler_params=_CP,
    )
    def image_kernel(xs_hbm, table_hbm, out_hbm, *, xs_v,
                     ib0, ib1, ov, fxv, fyv, t0, t1, out_v, sem):
        wid = lax.axis_index("s") * NC + lax.axis_index("c")
        base0 = wid * per_worker
        wf = jnp.float32(w)
        hf = jnp.float32(h)

        def chunk_body(ci, carry):
            base = base0 + ci * CHUNK
            pltpu.sync_copy(xs_hbm.at[pl.ds(base * 2, CHUNK * 2)], xs_v)

            iota = lax.iota(jnp.int32, L)

            def phase1(i, carry):
                p = i * L + iota
                p2 = p * 2
                x = plsc.load_gather(xs_v, [p2])
                y = plsc.load_gather(xs_v, [p2 + 1])
                sx = x * wf
                sy = y * hf
                xi = sx.astype(jnp.int32)
                yi = sy.astype(jnp.int32)
                fx = sx - xi.astype(jnp.float32)
                fy = sy - yi.astype(jnp.float32)
                x0 = jnp.clip(xi, 0, w - 1)
                y0 = jnp.clip(yi, 0, h - 1)
                fx = jnp.where(x0 == w - 1, jnp.float32(1.0), fx)
                fy = jnp.where(y0 == h - 1, jnp.float32(1.0), fy)
                x0 = jnp.minimum(x0, w - 2)
                y0 = jnp.minimum(y0, h - 2)
                b0 = (y0 * w + x0) * c
                k0 = jnp.right_shift(b0, 3)
                sl = pl.ds(i * L, L)
                plsc.store_scatter(ib0, [p2], k0)
                plsc.store_scatter(ib0, [p2 + 1], k0 + 1)
                plsc.store_scatter(ib1, [p2], k0 + krow)
                plsc.store_scatter(ib1, [p2 + 1], k0 + krow + 1)
                ov[sl] = b0 & 7
                fxv[sl] = fx
                fyv[sl] = fy
                return carry

            lax.fori_loop(0, CHUNK // L, phase1, 0, unroll=2)

            descs = []
            for j in range(CHUNK * 2 // GFAN):
                sl = pl.ds(j * GFAN, GFAN)
                for iv, tv in ((ib0, t0), (ib1, t1)):
                    descs.append(
                        pltpu.async_copy(table_hbm.at[iv.at[sl]], tv.at[sl], sem))
            for d in descs:
                d.wait()

            def phase3(i, carry):
                p = i * L + iota
                pc = p * c
                p2 = p * 2
                sl = pl.ds(i * L, L)
                o = ov[sl]
                fx = fxv[sl]
                fy = fyv[sl]
                gx = 1.0 - fx
                gy = 1.0 - fy
                for ch in range(c):
                    qa = o + ch
                    qb = qa + c
                    ra = p2 + jnp.right_shift(qa, 3)
                    ca = qa & 7
                    rb = p2 + jnp.right_shift(qb, 3)
                    cb = qb & 7
                    top = (plsc.load_gather(t0, [ra, ca]) * gx
                           + plsc.load_gather(t0, [rb, cb]) * fx)
                    bot = (plsc.load_gather(t1, [ra, ca]) * gx
                           + plsc.load_gather(t1, [rb, cb]) * fx)
                    plsc.store_scatter(out_v, [pc + ch], top * gy + bot * fy)
                return carry

            lax.fori_loop(0, CHUNK // L, phase3, 0, unroll=2)

            pltpu.sync_copy(out_v, out_hbm.at[pl.ds(base * c, CHUNK * c)])
            return carry

        lax.fori_loop(0, n_chunks, chunk_body, 0)

    return image_kernel


@jax.jit
def kernel(xs, data):
    h, w, c = data.shape
    n = xs.shape[0]
    table = _make_relayout(h * w * c)(data.reshape(-1))
    out_flat = _make_kernel(n, h, w, c)(xs.reshape(-1), table)
    return out_flat.reshape(n, c)


# speedup vs baseline: 5.9980x; 3.8167x over previous
import functools

import jax
import jax.numpy as jnp
from jax import lax
from jax.experimental import pallas as pl
from jax.experimental.pallas import tpu as pltpu
from jax.experimental.pallas import tpu_sc as plsc

NC = 2
NS = 16
NW = NC * NS
L = 16

CHUNK = 1024
GFAN = 128
TW = 8


def _make_kernel(n, h, w, c):
    per_worker = n // NW
    n_chunks = per_worker // CHUNK
    assert per_worker % CHUNK == 0 and CHUNK % (8 * L) == 0

    mesh = plsc.VectorSubcoreMesh(
        core_axis_name="c", subcore_axis_name="s",
        num_cores=NC, num_subcores=NS)

    @functools.partial(
        pl.kernel,
        out_type=jax.ShapeDtypeStruct((n * c,), jnp.float32),
        mesh=mesh,
        scratch_types=dict(
            xs_v=pltpu.VMEM((CHUNK * 2,), jnp.float32),
            i0=pltpu.VMEM((CHUNK,), jnp.int32),
            i1=pltpu.VMEM((CHUNK,), jnp.int32),
            fxv=pltpu.VMEM((CHUNK,), jnp.float32),
            fyv=pltpu.VMEM((CHUNK,), jnp.float32),
            t0=pltpu.VMEM((CHUNK, TW), jnp.float32),
            t1=pltpu.VMEM((CHUNK, TW), jnp.float32),
            out_v=pltpu.VMEM((CHUNK * c,), jnp.float32),
            sem=pltpu.SemaphoreType.DMA,
        ),
        compiler_params=pltpu.CompilerParams(
            needs_layout_passes=False, use_tc_tiling_on_sc=False),
    )
    def image_kernel(xs_hbm, table_hbm, out_hbm, *, xs_v,
                     i0, i1, fxv, fyv, t0, t1, out_v, sem):
        wid = lax.axis_index("s") * NC + lax.axis_index("c")
        base0 = wid * per_worker
        wf = jnp.float32(w)
        hf = jnp.float32(h)

        def chunk_body(ci, carry):
            base = base0 + ci * CHUNK
            pltpu.sync_copy(xs_hbm.at[pl.ds(base * 2, CHUNK * 2)], xs_v)

            iota = lax.iota(jnp.int32, L)

            def phase1(i, carry):
                p2 = (i * L + iota) * 2
                x = plsc.load_gather(xs_v, [p2])
                y = plsc.load_gather(xs_v, [p2 + 1])
                sx = x * wf
                sy = y * hf
                xi = sx.astype(jnp.int32)
                yi = sy.astype(jnp.int32)
                fx = sx - xi.astype(jnp.float32)
                fy = sy - yi.astype(jnp.float32)
                x0 = jnp.clip(xi, 0, w - 1)
                y0 = jnp.clip(yi, 0, h - 1)
                fx = jnp.where(x0 == w - 1, jnp.float32(1.0), fx)
                fy = jnp.where(y0 == h - 1, jnp.float32(1.0), fy)
                x0 = jnp.minimum(x0, w - 2)
                y0 = jnp.minimum(y0, h - 2)
                b0 = y0 * w + x0
                sl = pl.ds(i * L, L)
                i0[sl] = b0
                i1[sl] = b0 + w
                fxv[sl] = fx
                fyv[sl] = fy
                return carry

            lax.fori_loop(0, CHUNK // L, phase1, 0, unroll=2)

            descs = []
            for j in range(CHUNK // GFAN):
                sl = pl.ds(j * GFAN, GFAN)
                for iv, tv in ((i0, t0), (i1, t1)):
                    descs.append(
                        pltpu.async_copy(table_hbm.at[iv.at[sl]], tv.at[sl], sem))
            for d in descs:
                d.wait()

            def phase3(i, carry):
                p = i * L + iota
                pc = p * c
                sl = pl.ds(i * L, L)
                fx = fxv[sl]
                fy = fyv[sl]
                gx = 1.0 - fx
                gy = 1.0 - fy
                for ch in range(c):
                    cv = jnp.full((L,), ch, jnp.int32)
                    cv3 = jnp.full((L,), ch + c, jnp.int32)
                    top = (plsc.load_gather(t0, [p, cv]) * gx
                           + plsc.load_gather(t0, [p, cv3]) * fx)
                    bot = (plsc.load_gather(t1, [p, cv]) * gx
                           + plsc.load_gather(t1, [p, cv3]) * fx)
                    plsc.store_scatter(out_v, [pc + cv], top * gy + bot * fy)
                return carry

            lax.fori_loop(0, CHUNK // L, phase3, 0, unroll=2)

            pltpu.sync_copy(out_v, out_hbm.at[pl.ds(base * c, CHUNK * c)])
            return carry

        lax.fori_loop(0, n_chunks, chunk_body, 0)

    return image_kernel


@jax.jit
def kernel(xs, data):
    h, w, c = data.shape
    n = xs.shape[0]
    table = data.reshape(h * w, c)
    nxt = jnp.concatenate([table[1:], table[:1]], axis=0)
    pair = jnp.concatenate(
        [table, nxt, jnp.zeros((h * w, TW - 2 * c), table.dtype)], axis=1)
    out_flat = _make_kernel(n, h, w, c)(xs.reshape(-1), pair)
    return out_flat.reshape(n, c)
